# baseline (speedup 1.0000x reference)
import functools

import jax
import jax.numpy as jnp
from jax import lax
from jax.experimental import pallas as pl
from jax.experimental.pallas import tpu as pltpu
from jax.experimental.pallas import tpu_sc as plsc

_HEADS, _LEVELS, _POINTS, _C = 8, 4, 4, 256
_D = _C // _HEADS
_SHAPES = ((64, 64), (32, 32), (16, 16), (8, 8))
_STARTS = (0, 4096, 5120, 5376)
_LQ = 5440
_LQP = 5632
_QB = 512


def _proj_body(q_ref, x_ref, wv_ref, bv_ref, wso_ref, bso_ref, waw_ref, baw_ref,
               val_ref, so_ref, aw_ref):
    pad = ((0, 0), (0, _LQP - _LQ))
    x = x_ref[0]
    v = lax.dot_general(wv_ref[...], x, (((1,), (1,)), ((), ())),
                        preferred_element_type=jnp.float32)
    val_ref[0] = jnp.pad(v + bv_ref[...], pad)
    q = q_ref[0]
    so = lax.dot_general(wso_ref[...], q, (((1,), (1,)), ((), ())),
                         preferred_element_type=jnp.float32)
    so_ref[0] = jnp.pad(so + bso_ref[...], pad)
    a = lax.dot_general(waw_ref[...], q, (((1,), (1,)), ((), ())),
                        preferred_element_type=jnp.float32) + baw_ref[...]
    g = _LEVELS * _POINTS
    for h in range(_HEADS):
        blk = a[h * g:(h + 1) * g, :]
        m = jnp.max(blk, axis=0, keepdims=True)
        e = jnp.exp(blk - m)
        s = jnp.sum(e, axis=0, keepdims=True)
        aw_ref[0, h * g:(h + 1) * g, :] = jnp.pad(e / s, pad)


def _out_body(s_ref, wo_ref, bo_ref, o_ref):
    o = lax.dot_general(s_ref[0], wo_ref[...], (((0,), (1,)), ((), ())),
                        preferred_element_type=jnp.float32)
    o_ref[0] = o[:_LQ, :] + bo_ref[...]


def _sampler_body(val_hbm, rp_hbm, so_hbm, aw_hbm, out_hbm,
                  planes, rp_v, so_v, aw_v, out_v):
    wid = lax.axis_index("c") * 16 + lax.axis_index("s")
    n = wid // 16
    rem = wid % 16
    h = rem // 2
    grp = rem % 2
    row0 = h * _D + grp * 16
    nso = 2 * _LEVELS * _POINTS
    naw = _LEVELS * _POINTS

    pltpu.sync_copy(val_hbm.at[n, pl.ds(row0, 16), :], planes)

    ch_ids = [jnp.full((16,), ch, jnp.int32) for ch in range(16)]

    def block_body(b, _):
        q0 = b * _QB
        pltpu.sync_copy(rp_hbm.at[n, :, :, pl.ds(q0, _QB)], rp_v)
        pltpu.sync_copy(so_hbm.at[n, pl.ds(h * nso, nso), pl.ds(q0, _QB)], so_v)
        pltpu.sync_copy(aw_hbm.at[n, pl.ds(h * naw, naw), pl.ds(q0, _QB)], aw_v)

        @plsc.parallel_loop(0, _QB // 16, unroll=2)
        def _(k):
            qs = k * 16
            acc = [jnp.zeros((16,), jnp.float32) for _ in range(16)]
            for l in range(_LEVELS):
                hgt, wid_ = _SHAPES[l]
                base = _STARTS[l]
                rpx = rp_v[l, 0, pl.ds(qs, 16)]
                rpy = rp_v[l, 1, pl.ds(qs, 16)]
                for p in range(_POINTS):
                    r = l * 2 * _POINTS + p * 2
                    sox = so_v[r, pl.ds(qs, 16)]
                    soy = so_v[r + 1, pl.ds(qs, 16)]
                    a_w = aw_v[l * _POINTS + p, pl.ds(qs, 16)]
                    x = rpx + sox
                    y = rpy + soy
                    xt = x.astype(jnp.int32)
                    yt = y.astype(jnp.int32)
                    x0 = jnp.where(xt.astype(jnp.float32) > x, xt - 1, xt)
                    y0 = jnp.where(yt.astype(jnp.float32) > y, yt - 1, yt)
                    fx = x - x0.astype(jnp.float32)
                    fy = y - y0.astype(jnp.float32)
                    for dy in (0, 1):
                        yi = y0 + dy
                        wy = (fy if dy else 1.0 - fy) * a_w
                        vy = (yi >= 0) & (yi < hgt)
                        rowbase = base + yi * wid_
                        for dx in (0, 1):
                            xi = x0 + dx
                            wx = fx if dx else 1.0 - fx
                            ok = vy & (xi >= 0) & (xi < wid_)
                            wgt = wy * wx
                            pos = rowbase + xi
                            for ch in range(16):
                                vals = plsc.load_gather(planes, [ch_ids[ch], pos],
                                                        mask=ok)
                                acc[ch] = acc[ch] + wgt * vals
            for ch in range(16):
                out_v[ch, pl.ds(qs, 16)] = acc[ch]

        pltpu.sync_copy(out_v, out_hbm.at[n, pl.ds(row0, 16), pl.ds(q0, _QB)])
        return 0

    lax.fori_loop(0, _LQP // _QB, block_body, 0)


def kernel(query, reference_points, input_flatten, input_spatial_shapes,
           input_level_start_index, Wv, bv, Wso, bso, Waw, baw, Wo, bo):
    N, Lq, C = query.shape
    Len = input_flatten.shape[1]
    f32 = jnp.float32

    val_t, so_t, aw_t = pl.pallas_call(
        _proj_body,
        grid=(N,),
        in_specs=[
            pl.BlockSpec((1, Lq, C), lambda i: (i, 0, 0)),
            pl.BlockSpec((1, Len, C), lambda i: (i, 0, 0)),
            pl.BlockSpec((C, C), lambda i: (0, 0)),
            pl.BlockSpec((C, 1), lambda i: (0, 0)),
            pl.BlockSpec((C, C), lambda i: (0, 0)),
            pl.BlockSpec((C, 1), lambda i: (0, 0)),
            pl.BlockSpec((_HEADS * _LEVELS * _POINTS, C), lambda i: (0, 0)),
            pl.BlockSpec((_HEADS * _LEVELS * _POINTS, 1), lambda i: (0, 0)),
        ],
        out_specs=[
            pl.BlockSpec((1, C, _LQP), lambda i: (i, 0, 0)),
            pl.BlockSpec((1, C, _LQP), lambda i: (i, 0, 0)),
            pl.BlockSpec((1, _HEADS * _LEVELS * _POINTS, _LQP), lambda i: (i, 0, 0)),
        ],
        out_shape=[
            jax.ShapeDtypeStruct((N, C, _LQP), f32),
            jax.ShapeDtypeStruct((N, C, _LQP), f32),
            jax.ShapeDtypeStruct((N, _HEADS * _LEVELS * _POINTS, _LQP), f32),
        ],
    )(query, input_flatten, Wv, bv.reshape(C, 1), Wso, bso.reshape(-1, 1) - 0.5,
      Waw, baw.reshape(-1, 1))

    scale = jnp.asarray([[w_, h_] for h_, w_ in _SHAPES], f32)
    rp_t = jnp.pad(jnp.transpose(reference_points * scale[None, None], (0, 2, 3, 1)),
                   ((0, 0), (0, 0), (0, 0), (0, _LQP - _LQ)))

    mesh = plsc.VectorSubcoreMesh(core_axis_name="c", subcore_axis_name="s")
    sampler = functools.partial(
        pl.kernel,
        out_type=jax.ShapeDtypeStruct((N, C, _LQP), f32),
        mesh=mesh,
        compiler_params=pltpu.CompilerParams(use_tc_tiling_on_sc=False,
                                             needs_layout_passes=False),
        scratch_types=[
            pltpu.VMEM((16, _LQP), f32),
            pltpu.VMEM((_LEVELS, 2, _QB), f32),
            pltpu.VMEM((2 * _LEVELS * _POINTS, _QB), f32),
            pltpu.VMEM((_LEVELS * _POINTS, _QB), f32),
            pltpu.VMEM((16, _QB), f32),
        ],
    )(_sampler_body)
    sampled = sampler(val_t, rp_t, so_t, aw_t)

    out = pl.pallas_call(
        _out_body,
        grid=(N,),
        in_specs=[
            pl.BlockSpec((1, C, _LQP), lambda i: (i, 0, 0)),
            pl.BlockSpec((C, C), lambda i: (0, 0)),
            pl.BlockSpec((1, C), lambda i: (0, 0)),
        ],
        out_specs=pl.BlockSpec((1, Lq, C), lambda i: (i, 0, 0)),
        out_shape=jax.ShapeDtypeStruct((N, Lq, C), f32),
    )(sampled, Wo, bo.reshape(1, C))
    return out


# speedup vs baseline: 1.7389x; 1.7389x over previous
import functools

import jax
import jax.numpy as jnp
from jax import lax
from jax.experimental import pallas as pl
from jax.experimental.pallas import tpu as pltpu
from jax.experimental.pallas import tpu_sc as plsc

_HEADS, _LEVELS, _POINTS, _C = 8, 4, 4, 256
_D = _C // _HEADS
_SHAPES = ((64, 64), (32, 32), (16, 16), (8, 8))
_STARTS = (0, 4096, 5120, 5376)
_LQ = 5440
_LQP = 5632
_QB = 256


def _proj_body(q_ref, x_ref, wv_ref, bv_ref, wso_ref, bso_ref, waw_ref, baw_ref,
               val_ref, so_ref, aw_ref):
    pad = ((0, 0), (0, _LQP - _LQ))
    x = x_ref[0]
    v = lax.dot_general(wv_ref[...], x, (((1,), (1,)), ((), ())),
                        preferred_element_type=jnp.float32)
    val_ref[0] = jnp.pad(v + bv_ref[...], pad)
    q = q_ref[0]
    so = lax.dot_general(wso_ref[...], q, (((1,), (1,)), ((), ())),
                         preferred_element_type=jnp.float32)
    so_ref[0] = jnp.pad(so + bso_ref[...], pad)
    a = lax.dot_general(waw_ref[...], q, (((1,), (1,)), ((), ())),
                        preferred_element_type=jnp.float32) + baw_ref[...]
    g = _LEVELS * _POINTS
    for h in range(_HEADS):
        blk = a[h * g:(h + 1) * g, :]
        m = jnp.max(blk, axis=0, keepdims=True)
        e = jnp.exp(blk - m)
        s = jnp.sum(e, axis=0, keepdims=True)
        aw_ref[0, h * g:(h + 1) * g, :] = jnp.pad(e / s, pad)


def _out_body(s_ref, wo_ref, bo_ref, o_ref):
    o = lax.dot_general(s_ref[0], wo_ref[...], (((0,), (1,)), ((), ())),
                        preferred_element_type=jnp.float32)
    o_ref[0] = o[:_LQ, :] + bo_ref[...]


def _sampler_body(val_hbm, rp_hbm, so_hbm, aw_hbm, out_hbm,
                  planes, rp2, so2, aw2, out2, sin0, sin1, sout0, sout1):
    wid = lax.axis_index("c") * 16 + lax.axis_index("s")
    n = wid // 16
    rem = wid % 16
    h = rem // 2
    grp = rem % 2
    row0 = h * _D + grp * 16
    nso = 2 * _LEVELS * _POINTS
    naw = _LEVELS * _POINTS
    nblk = _LQP // _QB
    sins = (sin0, sin1)
    souts = (sout0, sout1)

    pltpu.sync_copy(val_hbm.at[n, pl.ds(row0, 16), :], planes)

    ch_ids = [jnp.full((16,), ch, jnp.int32) for ch in range(16)]

    def in_copies(bi, slot):
        q0 = jnp.minimum(bi, nblk - 1) * _QB
        return (
            (rp_hbm.at[n, :, :, pl.ds(q0, _QB)], rp2.at[slot], sins[slot]),
            (so_hbm.at[n, pl.ds(h * nso, nso), pl.ds(q0, _QB)], so2.at[slot], sins[slot]),
            (aw_hbm.at[n, pl.ds(h * naw, naw), pl.ds(q0, _QB)], aw2.at[slot], sins[slot]),
        )

    def prefetch(bi, slot):
        for src, dst, sem in in_copies(bi, slot):
            pltpu.async_copy(src, dst, sem)

    def wait_in(bi, slot):
        for src, dst, sem in in_copies(bi, slot):
            pltpu.make_async_copy(src, dst, sem).wait()

    def out_copy(b, slot):
        return (out2.at[slot],
                out_hbm.at[n, pl.ds(row0, 16),
                           pl.ds(jnp.minimum(b, nblk - 1) * _QB, _QB)],
                souts[slot])

    prefetch(0, 0)
    prefetch(1, 1)

    def compute_block(slot):
        rp_v, so_v, aw_v, out_v = rp2.at[slot], so2.at[slot], aw2.at[slot], out2.at[slot]

        @plsc.parallel_loop(0, _QB // 16)
        def _(k):
            qs = k * 16
            acc = [jnp.zeros((16,), jnp.float32) for _ in range(16)]
            for l in range(_LEVELS):
                hgt, wid_ = _SHAPES[l]
                base = _STARTS[l]
                rpx = rp_v[l, 0, pl.ds(qs, 16)]
                rpy = rp_v[l, 1, pl.ds(qs, 16)]
                for p in range(_POINTS):
                    r = l * 2 * _POINTS + p * 2
                    sox = so_v[r, pl.ds(qs, 16)]
                    soy = so_v[r + 1, pl.ds(qs, 16)]
                    a_w = aw_v[l * _POINTS + p, pl.ds(qs, 16)]
                    x = rpx + sox
                    y = rpy + soy
                    xt = x.astype(jnp.int32)
                    yt = y.astype(jnp.int32)
                    x0 = jnp.where(xt.astype(jnp.float32) > x, xt - 1, xt)
                    y0 = jnp.where(yt.astype(jnp.float32) > y, yt - 1, yt)
                    fx = x - x0.astype(jnp.float32)
                    fy = y - y0.astype(jnp.float32)
                    for dy in (0, 1):
                        yi = y0 + dy
                        wy = (fy if dy else 1.0 - fy) * a_w
                        vy = (yi >= 0) & (yi < hgt)
                        rowbase = base + yi * wid_
                        for dx in (0, 1):
                            xi = x0 + dx
                            wx = fx if dx else 1.0 - fx
                            ok = vy & (xi >= 0) & (xi < wid_)
                            wgt = wy * wx
                            pos = rowbase + xi
                            for ch in range(16):
                                vals = plsc.load_gather(planes, [ch_ids[ch], pos],
                                                        mask=ok)
                                acc[ch] = acc[ch] + wgt * vals
            for ch in range(16):
                out_v[ch, pl.ds(qs, 16)] = acc[ch]

    def pair_body(j, _):
        for slot in (0, 1):
            b = j * 2 + slot
            wait_in(b, slot)

            @pl.when(j > 0)
            def _():
                src, dst, sem = out_copy(b, slot)
                pltpu.make_async_copy(src, dst, sem).wait()

            compute_block(slot)
            src, dst, sem = out_copy(b, slot)
            pltpu.async_copy(src, dst, sem)
            prefetch(j * 2 + 2 + slot, slot)
        return 0

    lax.fori_loop(0, nblk // 2, pair_body, 0)
    for slot in (0, 1):
        wait_in(nblk, slot)
        src, dst, sem = out_copy(nblk - 2 + slot, slot)
        pltpu.make_async_copy(src, dst, sem).wait()


def kernel(query, reference_points, input_flatten, input_spatial_shapes,
           input_level_start_index, Wv, bv, Wso, bso, Waw, baw, Wo, bo):
    N, Lq, C = query.shape
    Len = input_flatten.shape[1]
    f32 = jnp.float32

    val_t, so_t, aw_t = pl.pallas_call(
        _proj_body,
        grid=(N,),
        in_specs=[
            pl.BlockSpec((1, Lq, C), lambda i: (i, 0, 0)),
            pl.BlockSpec((1, Len, C), lambda i: (i, 0, 0)),
            pl.BlockSpec((C, C), lambda i: (0, 0)),
            pl.BlockSpec((C, 1), lambda i: (0, 0)),
            pl.BlockSpec((C, C), lambda i: (0, 0)),
            pl.BlockSpec((C, 1), lambda i: (0, 0)),
            pl.BlockSpec((_HEADS * _LEVELS * _POINTS, C), lambda i: (0, 0)),
            pl.BlockSpec((_HEADS * _LEVELS * _POINTS, 1), lambda i: (0, 0)),
        ],
        out_specs=[
            pl.BlockSpec((1, C, _LQP), lambda i: (i, 0, 0)),
            pl.BlockSpec((1, C, _LQP), lambda i: (i, 0, 0)),
            pl.BlockSpec((1, _HEADS * _LEVELS * _POINTS, _LQP), lambda i: (i, 0, 0)),
        ],
        out_shape=[
            jax.ShapeDtypeStruct((N, C, _LQP), f32),
            jax.ShapeDtypeStruct((N, C, _LQP), f32),
            jax.ShapeDtypeStruct((N, _HEADS * _LEVELS * _POINTS, _LQP), f32),
        ],
    )(query, input_flatten, Wv, bv.reshape(C, 1), Wso, bso.reshape(-1, 1) - 0.5,
      Waw, baw.reshape(-1, 1))

    scale = jnp.asarray([[w_, h_] for h_, w_ in _SHAPES], f32)
    rp_t = jnp.pad(jnp.transpose(reference_points * scale[None, None], (0, 2, 3, 1)),
                   ((0, 0), (0, 0), (0, 0), (0, _LQP - _LQ)))

    mesh = plsc.VectorSubcoreMesh(core_axis_name="c", subcore_axis_name="s")
    sampler = functools.partial(
        pl.kernel,
        out_type=jax.ShapeDtypeStruct((N, C, _LQP), f32),
        mesh=mesh,
        compiler_params=pltpu.CompilerParams(use_tc_tiling_on_sc=False,
                                             needs_layout_passes=False),
        scratch_types=[
            pltpu.VMEM((16, _LQP), f32),
            pltpu.VMEM((2, _LEVELS, 2, _QB), f32),
            pltpu.VMEM((2, 2 * _LEVELS * _POINTS, _QB), f32),
            pltpu.VMEM((2, _LEVELS * _POINTS, _QB), f32),
            pltpu.VMEM((2, 16, _QB), f32),
            pltpu.SemaphoreType.DMA,
            pltpu.SemaphoreType.DMA,
            pltpu.SemaphoreType.DMA,
            pltpu.SemaphoreType.DMA,
        ],
    )(_sampler_body)
    sampled = sampler(val_t, rp_t, so_t, aw_t)

    out = pl.pallas_call(
        _out_body,
        grid=(N,),
        in_specs=[
            pl.BlockSpec((1, C, _LQP), lambda i: (i, 0, 0)),
            pl.BlockSpec((C, C), lambda i: (0, 0)),
            pl.BlockSpec((1, C), lambda i: (0, 0)),
        ],
        out_specs=pl.BlockSpec((1, Lq, C), lambda i: (i, 0, 0)),
        out_shape=jax.ShapeDtypeStruct((N, Lq, C), f32),
    )(sampled, Wo, bo.reshape(1, C))
    return out
